# baseline (speedup 1.0000x reference)
import functools
import jax
import jax.numpy as jnp
from jax import lax
from jax.experimental import pallas as pl
from jax.experimental.pallas import tpu as pltpu
from jax.experimental.pallas import tpu_sc as plsc

_NC = 2
_NS = 16
_NW = _NC * _NS
_D = 64
_SCALE = 8.0
_CHUNK = 40
_NBUF = 4
_TCR = 512


def _widen_scale(weight_t):
    V = weight_t.shape[1]

    def body(w_ref, o_ref):
        t = w_ref[...].T * _SCALE
        o_ref[...] = jnp.concatenate([t, t], axis=-1)

    return pl.pallas_call(
        body,
        grid=(pl.cdiv(V, _TCR),),
        in_specs=[pl.BlockSpec((_D, _TCR), lambda i: (0, i))],
        out_specs=pl.BlockSpec((_TCR, 2 * _D), lambda i: (i, 0)),
        out_shape=jax.ShapeDtypeStruct((V, 2 * _D), jnp.float32),
        compiler_params=pltpu.CompilerParams(
            dimension_semantics=("arbitrary",)),
    )(weight_t)


def _make_gather(bsz, seq, V):
    B = bsz * seq
    bpw = B // _NW
    nchunk = bpw // _CHUNK
    ipw = bsz // _NW
    cpi = seq // _CHUNK

    mesh = plsc.VectorSubcoreMesh(
        core_axis_name="c", subcore_axis_name="s",
        num_cores=_NC, num_subcores=_NS)

    @functools.partial(
        pl.kernel,
        out_type=jax.ShapeDtypeStruct((bsz, seq, _D), jnp.float32),
        mesh=mesh,
        scratch_types=[
            pltpu.VMEM((bpw,), jnp.int32),
            [pltpu.VMEM((_CHUNK, 2 * _D), jnp.float32)] * _NBUF,
            [pltpu.VMEM((_CHUNK, _D), jnp.float32)] * _NBUF,
            [pltpu.SemaphoreType.DMA] * _NBUF,
            [pltpu.SemaphoreType.DMA] * _NBUF,
        ],
    )
    def emb(idx_hbm, table_hbm, out_hbm, idx_v, bufs, obufs, gsems, osems):
        wid = lax.axis_index("s") * _NC + lax.axis_index("c")
        item0 = wid * ipw
        pltpu.sync_copy(idx_hbm.at[wid], idx_v)

        def fire_gather(j, b):
            pltpu.async_copy(
                table_hbm.at[idx_v.at[pl.ds(j * _CHUNK, _CHUNK)]],
                bufs[b], gsems[b])

        def wait_gather(j, b):
            pltpu.make_async_copy(
                table_hbm.at[idx_v.at[pl.ds(j * _CHUNK, _CHUNK)]],
                bufs[b], gsems[b]).wait()

        def out_slice(j):
            return out_hbm.at[item0 + j // cpi,
                              pl.ds((j % cpi) * _CHUNK, _CHUNK), :]

        for b in range(_NBUF - 1):
            fire_gather(b, b)

        @pl.loop(0, nchunk, step=_NBUF)
        def step(c):
            for db in range(_NBUF):
                j = c + db
                slot = db
                pb = (db + _NBUF - 1) % _NBUF
                wait_gather(j, slot)

                @pl.loop(0, _CHUNK)
                def extract(r):
                    for u in range(_D // 16):
                        s = pl.ds(u * 16, 16)
                        obufs[slot][r, s] = bufs[slot][r, s]

                pltpu.async_copy(obufs[slot], out_slice(j), osems[slot])

                @pl.when(j + _NBUF - 1 < nchunk)
                def _():
                    @pl.when(j >= 1)
                    def _():
                        pltpu.make_async_copy(
                            obufs[pb], out_slice(j - 1), osems[pb]).wait()
                    fire_gather(j + _NBUF - 1, pb)

        for j in range(nchunk - _NBUF, nchunk):
            slot = j % _NBUF
            pltpu.make_async_copy(
                obufs[slot], out_slice(j), osems[slot]).wait()

    return emb


def kernel(batch_inputs, weight):
    bsz, seq = batch_inputs.shape
    V = weight.shape[0]
    wide = _widen_scale(weight.T)
    idx = batch_inputs.astype(jnp.int32).reshape(_NW, (bsz * seq) // _NW)
    return _make_gather(bsz, seq, V)(idx, wide)


# speedup vs baseline: 1.5014x; 1.5014x over previous
import functools
import jax
import jax.numpy as jnp
from jax import lax
from jax.experimental import pallas as pl
from jax.experimental.pallas import tpu as pltpu
from jax.experimental.pallas import tpu_sc as plsc

_NC = 2
_NS = 16
_NW = _NC * _NS
_D = 64
_SCALE = 8.0
_CHUNK = 40
_NBUF = 4
_TCR = 2048


def _widen_scale(weight_t):
    V = weight_t.shape[1]

    def body(w_ref, o_ref):
        x = w_ref[...]
        eye = (lax.broadcasted_iota(jnp.int32, (_D, _D), 0)
               == lax.broadcasted_iota(jnp.int32, (_D, _D), 1))
        scaled_eye = eye.astype(jnp.float32) * _SCALE
        t = lax.dot_general(x, scaled_eye, (((0,), (0,)), ((), ())),
                            precision=lax.Precision.HIGHEST)
        o_ref[...] = jnp.concatenate([t, t], axis=-1)

    return pl.pallas_call(
        body,
        grid=(pl.cdiv(V, _TCR),),
        in_specs=[pl.BlockSpec((_D, _TCR), lambda i: (0, i))],
        out_specs=pl.BlockSpec((_TCR, 2 * _D), lambda i: (i, 0)),
        out_shape=jax.ShapeDtypeStruct((V, 2 * _D), jnp.float32),
        compiler_params=pltpu.CompilerParams(
            dimension_semantics=("arbitrary",)),
    )(weight_t)


def _make_gather(bsz, seq, V):
    B = bsz * seq
    bpw = B // _NW
    nchunk = bpw // _CHUNK
    ipw = bsz // _NW
    cpi = seq // _CHUNK

    mesh = plsc.VectorSubcoreMesh(
        core_axis_name="c", subcore_axis_name="s",
        num_cores=_NC, num_subcores=_NS)

    @functools.partial(
        pl.kernel,
        out_type=jax.ShapeDtypeStruct((bsz, seq, _D), jnp.float32),
        mesh=mesh,
        scratch_types=[
            pltpu.VMEM((bpw,), jnp.int32),
            [pltpu.VMEM((_CHUNK, 2 * _D), jnp.float32)] * _NBUF,
            [pltpu.VMEM((_CHUNK, _D), jnp.float32)] * _NBUF,
            [pltpu.SemaphoreType.DMA] * _NBUF,
            [pltpu.SemaphoreType.DMA] * _NBUF,
        ],
    )
    def emb(idx_hbm, table_hbm, out_hbm, idx_v, bufs, obufs, gsems, osems):
        wid = lax.axis_index("s") * _NC + lax.axis_index("c")
        item0 = wid * ipw
        pltpu.sync_copy(idx_hbm.at[wid], idx_v)

        def fire_gather(j, b):
            pltpu.async_copy(
                table_hbm.at[idx_v.at[pl.ds(j * _CHUNK, _CHUNK)]],
                bufs[b], gsems[b])

        def wait_gather(j, b):
            pltpu.make_async_copy(
                table_hbm.at[idx_v.at[pl.ds(j * _CHUNK, _CHUNK)]],
                bufs[b], gsems[b]).wait()

        def out_slice(j):
            return out_hbm.at[item0 + j // cpi,
                              pl.ds((j % cpi) * _CHUNK, _CHUNK), :]

        for b in range(_NBUF - 1):
            fire_gather(b, b)

        @pl.loop(0, nchunk, step=_NBUF)
        def step(c):
            for db in range(_NBUF):
                j = c + db
                slot = db
                pb = (db + _NBUF - 1) % _NBUF
                wait_gather(j, slot)

                @pl.loop(0, _CHUNK)
                def extract(r):
                    for u in range(_D // 16):
                        s = pl.ds(u * 16, 16)
                        obufs[slot][r, s] = bufs[slot][r, s]

                pltpu.async_copy(obufs[slot], out_slice(j), osems[slot])

                @pl.when(j + _NBUF - 1 < nchunk)
                def _():
                    @pl.when(j >= 1)
                    def _():
                        pltpu.make_async_copy(
                            obufs[pb], out_slice(j - 1), osems[pb]).wait()
                    fire_gather(j + _NBUF - 1, pb)

        for j in range(nchunk - _NBUF, nchunk):
            slot = j % _NBUF
            pltpu.make_async_copy(
                obufs[slot], out_slice(j), osems[slot]).wait()

    return emb


def kernel(batch_inputs, weight):
    bsz, seq = batch_inputs.shape
    V = weight.shape[0]
    wide = _widen_scale(weight.T)
    idx = batch_inputs.astype(jnp.int32).reshape(_NW, (bsz * seq) // _NW)
    return _make_gather(bsz, seq, V)(idx, wide)
